# baseline (speedup 1.0000x reference)
import functools

import jax
import jax.numpy as jnp
from jax import lax
from jax.experimental import pallas as pl
from jax.experimental.pallas import tpu as pltpu
from jax.experimental.pallas import tpu_sc as plsc

_N = 1048576
_CELLS = 128
_NC, _NS, _L = 2, 16, 16
_NW = _NC * _NS
_PPW = _N // _NW
_C = 4096
_NCHUNK = _PPW // _C
_SUB = 128
_NSUB = _C // _SUB
_GPS = _SUB // _L

_mesh = plsc.VectorSubcoreMesh(core_axis_name="c", subcore_axis_name="s")


@functools.partial(
    pl.kernel,
    out_type=(
        jax.ShapeDtypeStruct((3 * _N,), jnp.float32),
        jax.ShapeDtypeStruct((_N,), jnp.float32),
    ),
    mesh=_mesh,
    compiler_params=pltpu.CompilerParams(
        needs_layout_passes=False, use_tc_tiling_on_sc=False),
    scratch_types=[
        pltpu.VMEM((3 * _C,), jnp.float32),
        pltpu.VMEM((_NSUB, _SUB), jnp.int32),
        pltpu.VMEM((_C,), jnp.float32),
        pltpu.VMEM((_C,), jnp.int32),
        pltpu.VMEM((_C, 8), jnp.float32),
        pltpu.VMEM((3 * _C,), jnp.float32),
        pltpu.VMEM((_C,), jnp.float32),
        pltpu.SemaphoreType.DMA,
    ],
)
def _voxel_sc(xyz_hbm, grid_hbm, rgb_hbm, den_hbm,
              xyz_v, idx_v, cond_v, sel_v, rows_v, rgb_v, den_v, sem):
    wid = lax.axis_index("s") * _NC + lax.axis_index("c")
    lanes = lax.iota(jnp.int32, _L)
    lanes3 = lanes * 3

    def to_cell(v):
        i = (v * jnp.float32(_CELLS) + jnp.float32(_CELLS // 2)).astype(jnp.int32)
        return jnp.clip(i, 0, _CELLS - 1)

    def chunk_body(ci, _):
        base = wid * _PPW + ci * _C
        pltpu.sync_copy(xyz_hbm.at[pl.ds(3 * base, 3 * _C)], xyz_v)

        def pass1(j, _):
            for t in range(_GPS):
                g16 = j * _SUB + t * _L
                i0 = lanes3 + g16 * 3
                x = plsc.load_gather(xyz_v, [i0])
                y = plsc.load_gather(xyz_v, [i0 + 1])
                z = plsc.load_gather(xyz_v, [i0 + 2])
                half = jnp.float32(0.5)
                cond = ((jnp.abs(x) < half) & (jnp.abs(y) < half)
                        & (jnp.abs(z) < half))
                fi = ((to_cell(x) << 14) + (to_cell(y) << 7) + to_cell(z))
                idx_v[j, pl.ds(t * _L, _L)] = fi >> 1
                sel_v[pl.ds(g16, _L)] = (fi & 1) << 2
                cond_v[pl.ds(g16, _L)] = jnp.where(cond, 1.0, 0.0).astype(jnp.float32)
            return 0

        lax.fori_loop(0, _NSUB, pass1, 0)

        copies = [
            pltpu.async_copy(grid_hbm.at[idx_v.at[j]],
                             rows_v.at[pl.ds(j * _SUB, _SUB)], sem)
            for j in range(_NSUB)
        ]
        for cp in copies:
            cp.wait()

        ch_splats = [jnp.full((_L,), c, jnp.int32) for c in range(4)]

        def pass2(j, _):
            for t in range(_GPS):
                g16 = j * _SUB + t * _L
                rowi = lanes + g16
                cf = cond_v[pl.ds(g16, _L)]
                half = sel_v[pl.ds(g16, _L)]
                vals = [plsc.load_gather(rows_v, [rowi, half + ch_splats[c]]) * cf
                        for c in range(4)]
                o3 = lanes3 + g16 * 3
                one = jnp.float32(1.0)
                for c in range(3):
                    s = one / (one + jnp.exp(-vals[c]))
                    plsc.store_scatter(rgb_v, [o3 + c], s)
                den_v[pl.ds(g16, _L)] = jnp.maximum(vals[3], 0.0)
            return 0

        lax.fori_loop(0, _NSUB, pass2, 0)

        pltpu.sync_copy(rgb_v, rgb_hbm.at[pl.ds(3 * base, 3 * _C)])
        pltpu.sync_copy(den_v, den_hbm.at[pl.ds(base, _C)])
        return 0

    lax.fori_loop(0, _NCHUNK, chunk_body, 0)


def kernel(xyz, grid):
    rgb_flat, den = _voxel_sc(xyz.reshape(3 * _N),
                              grid.reshape(_CELLS * _CELLS * _CELLS // 2, 8))
    return rgb_flat.reshape(_N, 3), den.reshape(_N, 1)


# speedup vs baseline: 1.6004x; 1.6004x over previous
import functools

import jax
import jax.numpy as jnp
from jax import lax
from jax.experimental import pallas as pl
from jax.experimental.pallas import tpu as pltpu
from jax.experimental.pallas import tpu_sc as plsc

_N = 1048576
_CELLS = 128
_NC, _NS, _L = 2, 16, 16
_NW = _NC * _NS
_PPW = _N // _NW
_C = 4096
_NCHUNK = _PPW // _C
_TPC = _C // 128

_mesh = plsc.VectorSubcoreMesh(core_axis_name="c", subcore_axis_name="s")


@functools.partial(
    pl.kernel,
    out_type=(
        jax.ShapeDtypeStruct((4 * _N,), jnp.float32),
        jax.ShapeDtypeStruct((_N,), jnp.float32),
    ),
    mesh=_mesh,
    compiler_params=pltpu.CompilerParams(
        needs_layout_passes=False, use_tc_tiling_on_sc=False),
    scratch_types=[
        pltpu.VMEM((3 * _C,), jnp.float32),
        pltpu.VMEM((_C,), jnp.int32),
        pltpu.VMEM((_C,), jnp.int32),
        pltpu.VMEM((_C,), jnp.int32),
        pltpu.VMEM((_C,), jnp.int32),
        pltpu.VMEM((_C,), jnp.float32),
        pltpu.VMEM((_C,), jnp.float32),
        pltpu.VMEM((_C,), jnp.float32),
        pltpu.VMEM((_C,), jnp.float32),
        pltpu.VMEM((_C,), jnp.float32),
        pltpu.VMEM((4 * _C,), jnp.float32),
        pltpu.VMEM((_C,), jnp.float32),
        pltpu.SemaphoreType.DMA,
    ],
)
def _voxel_sc(xyz_hbm, grid_hbm, rgb_hbm, den_hbm,
              xyz_v, ix0, ix1, ix2, ix3, v0, v1, v2, v3,
              cond_v, rgb_v, den_v, sem):
    wid = lax.axis_index("s") * _NC + lax.axis_index("c")
    lanes = lax.iota(jnp.int32, _L)
    lanes3 = lanes * 3
    idx_refs = (ix0, ix1, ix2, ix3)
    val_refs = (v0, v1, v2, v3)

    def to_cell(v):
        i = (v * jnp.float32(_CELLS) + jnp.float32(_CELLS // 2)).astype(jnp.int32)
        return jnp.clip(i, 0, _CELLS - 1)

    def chunk_body(ci, _):
        base = wid * _PPW + ci * _C
        pltpu.sync_copy(xyz_hbm.at[pl.ds(3 * base, 3 * _C)], xyz_v)

        def pass1(j, _):
            for t in range(8):
                g16 = j * 128 + t * _L
                i0 = lanes3 + g16 * 3
                x = plsc.load_gather(xyz_v, [i0])
                y = plsc.load_gather(xyz_v, [i0 + 1])
                z = plsc.load_gather(xyz_v, [i0 + 2])
                half = jnp.float32(0.5)
                cond = ((jnp.abs(x) < half) & (jnp.abs(y) < half)
                        & (jnp.abs(z) < half))
                e = (to_cell(x) * 128 + to_cell(y)) * 512 + to_cell(z)
                for c in range(4):
                    idx_refs[c][pl.ds(g16, _L)] = e + c * 128
                cond_v[pl.ds(g16, _L)] = jnp.where(cond, 1.0, 0.0).astype(jnp.float32)
            return 0

        lax.fori_loop(0, _TPC, pass1, 0)

        copies = [pltpu.async_copy(grid_hbm.at[idx_refs[c]], val_refs[c], sem)
                  for c in range(4)]
        for cp in copies:
            cp.wait()

        def pass2(j, _):
            for t in range(8):
                g16 = j * 128 + t * _L
                cf = cond_v[pl.ds(g16, _L)]
                one = jnp.float32(1.0)
                for c in range(3):
                    s = val_refs[c][pl.ds(g16, _L)] * cf
                    rgb_v[pl.ds(j * 512 + c * 128 + t * _L, _L)] = (
                        one / (one + jnp.exp(-s)))
                d = val_refs[3][pl.ds(g16, _L)] * cf
                den_v[pl.ds(g16, _L)] = jnp.maximum(d, 0.0)
            return 0

        lax.fori_loop(0, _TPC, pass2, 0)

        pltpu.sync_copy(rgb_v, rgb_hbm.at[pl.ds(4 * base, 4 * _C)])
        pltpu.sync_copy(den_v, den_hbm.at[pl.ds(base, _C)])
        return 0

    lax.fori_loop(0, _NCHUNK, chunk_body, 0)


def kernel(xyz, grid):
    grid_lin = grid.transpose(0, 1, 3, 2).reshape(-1)
    rgb4, den = _voxel_sc(xyz.reshape(3 * _N), grid_lin)
    rgb = rgb4.reshape(_N // 128, 4, 128)[:, :3, :].transpose(0, 2, 1)
    return rgb.reshape(_N, 3), den.reshape(_N, 1)
